# baseline (speedup 1.0000x reference)
import jax
import jax.numpy as jnp
from jax import lax
from jax.experimental import pallas as pl
from jax.experimental.pallas import tpu as pltpu
from jax.experimental.pallas import tpu_sc as plsc

P = 16
S = 512
T = 32768
L = 16
NW = 32

TC_T = 24576
SC_T = T - TC_T
BT = 4096

SC_ROWS_PER_TILE = SC_T // NW
CHUNK = 64
NCHUNK = SC_ROWS_PER_TILE // CHUNK
NPAIR = NCHUNK // 2

_LN2 = 0.6931471805599453
_SQRT2 = 1.4142135623730951

NACC = 4
UNROLL = 16


def _vlog(x):
    b = lax.bitcast_convert_type(x, jnp.int32)
    e = ((b >> 23) & 0xFF) - 127
    m = lax.bitcast_convert_type((b & 0x007FFFFF) | 0x3F800000, jnp.float32)
    big = m > _SQRT2
    e = jnp.where(big, e + 1, e)
    m = jnp.where(big, m * 0.5, m)
    r = (m - 1.0) / (m + 1.0)
    r2 = r * r
    poly = 1.0 + r2 * (0.3333333333 + r2 * (0.2 + r2 * 0.142857143))
    return e.astype(jnp.float32) * _LN2 + 2.0 * r * poly


def _group(buf, costs_v, valid_v, seg_v, out_v, local0, grp):
    off = local0 + grp * L
    seg_vec = seg_v[pl.ds(off, L)]
    lane = lax.iota(jnp.int32, L)
    row_vec = lane + grp * L
    col0 = (lane * 33) & (S - 1)

    def body(c, carry):
        accs, col = carry
        accs = list(accs)
        for u in range(UNROLL):
            a = plsc.load_gather(buf, [row_vec, col])
            w = plsc.load_gather(costs_v, [seg_vec, col])
            accs[u % NACC] = accs[u % NACC] + a * w
            col = (col + 1) & (S - 1)
        return tuple(accs), col

    init = tuple(jnp.zeros((L,), jnp.float32) for _ in range(NACC))
    accs, _ = lax.fori_loop(0, S // UNROLL, body, (init, col0))
    res = (accs[0] + accs[1]) + (accs[2] + accs[3])

    vq = plsc.load_gather(valid_v, [seg_vec]) > 0
    s_safe = jnp.where(vq, res, 1.0)
    out_v[pl.ds(off, L)] = jnp.where(vq, _vlog(s_safe), 0.0)


def _sc_body(occ_hbm, costs_hbm, valid_hbm, seg_hbm, out_hbm,
             costs_v, valid_v, seg_v, out_v, buf0, buf1, sem0, sem1):
    wid = lax.axis_index("s") * 2 + lax.axis_index("c")
    base = TC_T + wid * SC_ROWS_PER_TILE

    pltpu.sync_copy(costs_hbm, costs_v)
    pltpu.sync_copy(valid_hbm, valid_v)
    pltpu.sync_copy(seg_hbm.at[pl.ds(base, SC_ROWS_PER_TILE)], seg_v)

    pltpu.async_copy(occ_hbm.at[pl.ds(base, CHUNK)], buf0, sem0)
    pltpu.async_copy(occ_hbm.at[pl.ds(base + CHUNK, CHUNK)], buf1, sem1)

    def pair(i, carry):
        c0 = 2 * i
        for b, (buf, sem) in enumerate(((buf0, sem0), (buf1, sem1))):
            cidx = c0 + b
            local0 = cidx * CHUNK
            pltpu.make_async_copy(occ_hbm.at[pl.ds(0, CHUNK)], buf, sem).wait()

            def do_group(g, c, buf=buf, local0=local0):
                _group(buf, costs_v, valid_v, seg_v, out_v, local0, g)
                return c

            lax.fori_loop(0, CHUNK // L, do_group, 0)

            @pl.when(cidx + 2 < NCHUNK)
            def _():
                pltpu.async_copy(
                    occ_hbm.at[pl.ds(base + (cidx + 2) * CHUNK, CHUNK)],
                    buf, sem)

        return carry

    lax.fori_loop(0, NPAIR, pair, 0)
    if NCHUNK % 2:
        cidx = NCHUNK - 1
        local0 = cidx * CHUNK
        pltpu.make_async_copy(occ_hbm.at[pl.ds(0, CHUNK)], buf0, sem0).wait()

        def do_last(g, c):
            _group(buf0, costs_v, valid_v, seg_v, out_v, local0, g)
            return c

        lax.fori_loop(0, CHUNK // L, do_last, 0)
    pltpu.sync_copy(
        out_v, out_hbm.at[pl.ds(wid * SC_ROWS_PER_TILE, SC_ROWS_PER_TILE)])


def _tc_body(occ_ref, costs_ref, valid_ref, seg_ref, out_ref):
    x = occ_ref[...]
    c = costs_ref[...]
    m = jax.lax.dot_general(
        x, c, (((1,), (1,)), ((), ())),
        preferred_element_type=jnp.float32)
    seg = seg_ref[0, 0, :]
    bt = x.shape[0]
    pid = jax.lax.broadcasted_iota(jnp.int32, (bt, P), 1)
    onehot = seg[:, None] == pid
    s = jnp.sum(jnp.where(onehot, m, 0.0), axis=1)
    vf = valid_ref[0, :]
    vq = jnp.sum(jnp.where(onehot, jnp.broadcast_to(vf[None, :], (bt, P)), 0.0),
                 axis=1) > 0.5
    out_ref[0, 0, :] = jnp.where(vq, jnp.log(jnp.where(vq, s, 1.0)), 0.0)


def kernel(occurrence_count, costs, valid, segment_ids):
    valid_i = valid.astype(jnp.int32)
    valid_f = valid.astype(jnp.float32).reshape(1, P)

    sc_run = pl.kernel(
        _sc_body,
        out_type=jax.ShapeDtypeStruct((SC_T,), jnp.float32),
        mesh=plsc.VectorSubcoreMesh(core_axis_name="c", subcore_axis_name="s"),
        compiler_params=pltpu.CompilerParams(
            needs_layout_passes=False, skip_device_barrier=True),
        scratch_types=[
            pltpu.VMEM((P, S), jnp.float32),
            pltpu.VMEM((P,), jnp.int32),
            pltpu.VMEM((SC_ROWS_PER_TILE,), jnp.int32),
            pltpu.VMEM((SC_ROWS_PER_TILE,), jnp.float32),
            pltpu.VMEM((CHUNK, S), jnp.float32),
            pltpu.VMEM((CHUNK, S), jnp.float32),
            pltpu.SemaphoreType.DMA,
            pltpu.SemaphoreType.DMA,
        ],
    )
    sc_out = sc_run(occurrence_count, costs, valid_i, segment_ids)

    nb = TC_T // BT
    seg3 = segment_ids.reshape(T // BT, 1, BT)
    tc_out = pl.pallas_call(
        _tc_body,
        grid=(nb,),
        in_specs=[
            pl.BlockSpec((BT, S), lambda i: (i, 0)),
            pl.BlockSpec((P, S), lambda i: (0, 0)),
            pl.BlockSpec((1, P), lambda i: (0, 0)),
            pl.BlockSpec((1, 1, BT), lambda i: (i, 0, 0)),
        ],
        out_specs=pl.BlockSpec((1, 1, BT), lambda i: (i, 0, 0)),
        out_shape=jax.ShapeDtypeStruct((nb, 1, BT), jnp.float32),
    )(occurrence_count, costs, valid_f, seg3)

    return jnp.concatenate([tc_out.reshape(TC_T), sc_out])


# speedup vs baseline: 1.0357x; 1.0357x over previous
import jax
import jax.numpy as jnp
from jax import lax
from jax.experimental import pallas as pl
from jax.experimental.pallas import tpu as pltpu
from jax.experimental.pallas import tpu_sc as plsc

P = 16
S = 512
T = 32768
L = 16
NW = 32

TC_T = 24576
SC_T = T - TC_T
BT = 2048

SC_ROWS_PER_TILE = SC_T // NW
CHUNK = 64
NCHUNK = SC_ROWS_PER_TILE // CHUNK
NPAIR = NCHUNK // 2

_LN2 = 0.6931471805599453
_SQRT2 = 1.4142135623730951

NACC = 4
UNROLL = 16


def _vlog(x):
    b = lax.bitcast_convert_type(x, jnp.int32)
    e = ((b >> 23) & 0xFF) - 127
    m = lax.bitcast_convert_type((b & 0x007FFFFF) | 0x3F800000, jnp.float32)
    big = m > _SQRT2
    e = jnp.where(big, e + 1, e)
    m = jnp.where(big, m * 0.5, m)
    r = (m - 1.0) / (m + 1.0)
    r2 = r * r
    poly = 1.0 + r2 * (0.3333333333 + r2 * (0.2 + r2 * 0.142857143))
    return e.astype(jnp.float32) * _LN2 + 2.0 * r * poly


def _group(buf, costs_v, valid_v, seg_v, out_v, local0, grp):
    off = local0 + grp * L
    seg_vec = seg_v[pl.ds(off, L)]
    lane = lax.iota(jnp.int32, L)
    row_vec = lane + grp * L
    col0 = (lane * 33) & (S - 1)

    def body(c, carry):
        accs, col = carry
        accs = list(accs)
        for u in range(UNROLL):
            a = plsc.load_gather(buf, [row_vec, col])
            w = plsc.load_gather(costs_v, [seg_vec, col])
            accs[u % NACC] = accs[u % NACC] + a * w
            col = (col + 1) & (S - 1)
        return tuple(accs), col

    init = tuple(jnp.zeros((L,), jnp.float32) for _ in range(NACC))
    accs, _ = lax.fori_loop(0, S // UNROLL, body, (init, col0))
    res = (accs[0] + accs[1]) + (accs[2] + accs[3])

    vq = plsc.load_gather(valid_v, [seg_vec]) > 0
    s_safe = jnp.where(vq, res, 1.0)
    out_v[pl.ds(off, L)] = jnp.where(vq, _vlog(s_safe), 0.0)


def _sc_body(occ_hbm, costs_hbm, valid_hbm, seg_hbm, out_hbm,
             costs_v, valid_v, seg_v, out_v, buf0, buf1, sem0, sem1, sem2):
    wid = lax.axis_index("s") * 2 + lax.axis_index("c")
    base = TC_T + wid * SC_ROWS_PER_TILE

    pltpu.async_copy(occ_hbm.at[pl.ds(base, CHUNK)], buf0, sem0)
    pltpu.async_copy(occ_hbm.at[pl.ds(base + CHUNK, CHUNK)], buf1, sem1)
    pltpu.async_copy(costs_hbm, costs_v, sem2)
    pltpu.async_copy(valid_hbm, valid_v, sem2)
    pltpu.async_copy(seg_hbm.at[pl.ds(base, SC_ROWS_PER_TILE)], seg_v, sem2)
    pltpu.make_async_copy(costs_hbm, costs_v, sem2).wait()
    pltpu.make_async_copy(valid_hbm, valid_v, sem2).wait()
    pltpu.make_async_copy(
        seg_hbm.at[pl.ds(base, SC_ROWS_PER_TILE)], seg_v, sem2).wait()

    def pair(i, carry):
        c0 = 2 * i
        for b, (buf, sem) in enumerate(((buf0, sem0), (buf1, sem1))):
            cidx = c0 + b
            local0 = cidx * CHUNK
            pltpu.make_async_copy(occ_hbm.at[pl.ds(0, CHUNK)], buf, sem).wait()

            def do_group(g, c, buf=buf, local0=local0):
                _group(buf, costs_v, valid_v, seg_v, out_v, local0, g)
                return c

            lax.fori_loop(0, CHUNK // L, do_group, 0)

            @pl.when(cidx + 2 < NCHUNK)
            def _():
                pltpu.async_copy(
                    occ_hbm.at[pl.ds(base + (cidx + 2) * CHUNK, CHUNK)],
                    buf, sem)

        return carry

    lax.fori_loop(0, NPAIR, pair, 0)
    if NCHUNK % 2:
        cidx = NCHUNK - 1
        local0 = cidx * CHUNK
        pltpu.make_async_copy(occ_hbm.at[pl.ds(0, CHUNK)], buf0, sem0).wait()

        def do_last(g, c):
            _group(buf0, costs_v, valid_v, seg_v, out_v, local0, g)
            return c

        lax.fori_loop(0, CHUNK // L, do_last, 0)
    pltpu.sync_copy(
        out_v, out_hbm.at[pl.ds(wid * SC_ROWS_PER_TILE, SC_ROWS_PER_TILE)])


def _tc_body(occ_ref, costs_ref, valid_ref, seg_ref, out_ref):
    x = occ_ref[...]
    c = costs_ref[...]
    m = jax.lax.dot_general(
        x, c, (((1,), (1,)), ((), ())),
        preferred_element_type=jnp.float32)
    seg = seg_ref[0, 0, :]
    bt = x.shape[0]
    pid = jax.lax.broadcasted_iota(jnp.int32, (bt, P), 1)
    onehot = seg[:, None] == pid
    s = jnp.sum(jnp.where(onehot, m, 0.0), axis=1)
    vf = valid_ref[0, :]
    vq = jnp.sum(jnp.where(onehot, jnp.broadcast_to(vf[None, :], (bt, P)), 0.0),
                 axis=1) > 0.5
    out_ref[0, 0, :] = jnp.where(vq, jnp.log(jnp.where(vq, s, 1.0)), 0.0)


def kernel(occurrence_count, costs, valid, segment_ids):
    valid_i = valid.astype(jnp.int32)
    valid_f = valid.astype(jnp.float32).reshape(1, P)

    sc_run = pl.kernel(
        _sc_body,
        out_type=jax.ShapeDtypeStruct((SC_T,), jnp.float32),
        mesh=plsc.VectorSubcoreMesh(core_axis_name="c", subcore_axis_name="s"),
        compiler_params=pltpu.CompilerParams(
            needs_layout_passes=False, skip_device_barrier=True),
        scratch_types=[
            pltpu.VMEM((P, S), jnp.float32),
            pltpu.VMEM((P,), jnp.int32),
            pltpu.VMEM((SC_ROWS_PER_TILE,), jnp.int32),
            pltpu.VMEM((SC_ROWS_PER_TILE,), jnp.float32),
            pltpu.VMEM((CHUNK, S), jnp.float32),
            pltpu.VMEM((CHUNK, S), jnp.float32),
            pltpu.SemaphoreType.DMA,
            pltpu.SemaphoreType.DMA,
            pltpu.SemaphoreType.DMA,
        ],
    )
    sc_out = sc_run(occurrence_count, costs, valid_i, segment_ids)

    nb = TC_T // BT
    seg3 = segment_ids.reshape(T // BT, 1, BT)
    tc_out = pl.pallas_call(
        _tc_body,
        grid=(nb,),
        in_specs=[
            pl.BlockSpec((BT, S), lambda i: (i, 0)),
            pl.BlockSpec((P, S), lambda i: (0, 0)),
            pl.BlockSpec((1, P), lambda i: (0, 0)),
            pl.BlockSpec((1, 1, BT), lambda i: (i, 0, 0)),
        ],
        out_specs=pl.BlockSpec((1, 1, BT), lambda i: (i, 0, 0)),
        out_shape=jax.ShapeDtypeStruct((nb, 1, BT), jnp.float32),
    )(occurrence_count, costs, valid_f, seg3)

    return jnp.concatenate([tc_out.reshape(TC_T), sc_out])
